# baseline (speedup 1.0000x reference)
import functools

import jax
import jax.numpy as jnp
from jax import lax
from jax.experimental import pallas as pl
from jax.experimental.pallas import tpu as pltpu
from jax.experimental.pallas import tpu_sc as plsc

N = 10000
E = 320000
D = 128
C = 8
NUM_LAYERS = 3

NCORE = 2
NSUB = 16
NW = NCORE * NSUB
K = 128
CH = 80
EPAD = NW * CH * K
BLK = 40
NPAD = 10240
STRIPE = NPAD // NSUB

R = 2048

_mesh = plsc.VectorSubcoreMesh(core_axis_name="c", subcore_axis_name="s")


@functools.partial(
    pl.kernel,
    mesh=_mesh,
    out_type=jax.ShapeDtypeStruct((NCORE, NPAD), jnp.float32),
    scratch_types=[
        pltpu.VMEM((CH, K), jnp.int32),
        pltpu.VMEM((K,), jnp.float32),
        pltpu.VMEM_SHARED((NPAD,), jnp.float32),
    ],
)
def _sc_degree(dst3, z1d, out, didx, ones_v, deg_sh):
    c = lax.axis_index("c")
    s = lax.axis_index("s")
    w = c * NSUB + s
    pltpu.sync_copy(dst3.at[w], didx)
    one16 = jnp.ones((16,), jnp.float32)
    for i in range(K // 16):
        ones_v[pl.ds(i * 16, 16)] = one16
    pltpu.sync_copy(z1d.at[pl.ds(s * STRIPE, STRIPE)],
                    deg_sh.at[pl.ds(s * STRIPE, STRIPE)])
    plsc.subcore_barrier()

    def body(j, carry):
        pltpu.sync_copy(ones_v, deg_sh.at[didx.at[j]], add=True)
        return carry

    lax.fori_loop(0, CH, body, 0)
    plsc.subcore_barrier()
    pltpu.sync_copy(deg_sh.at[pl.ds(s * STRIPE, STRIPE)],
                    out.at[c, pl.ds(s * STRIPE, STRIPE)])


def _make_propagate(fd):
    @functools.partial(
        pl.kernel,
        mesh=_mesh,
        out_type=jax.ShapeDtypeStruct((NCORE, NPAD, fd), jnp.float32),
        scratch_types=[
            pltpu.VMEM((BLK, K), jnp.int32),
            pltpu.VMEM((BLK, K), jnp.int32),
            pltpu.VMEM((K, fd), jnp.float32),
            pltpu.VMEM((K, fd), jnp.float32),
            pltpu.VMEM_SHARED((NPAD, fd), jnp.float32),
            pltpu.SemaphoreType.DMA,
            pltpu.SemaphoreType.DMA,
        ],
    )
    def _prop(m_hbm, src3, dst3, zrows, out, sidx, didx, msg0, msg1,
              acc_sh, sem0, sem1):
        _propagate_body(m_hbm, src3, dst3, zrows, out, sidx, didx, msg0, msg1,
                        acc_sh, sem0, sem1)

    return _prop


def _propagate_body(m_hbm, src3, dst3, zrows, out, sidx, didx, msg0, msg1,
                    acc_sh, sem0, sem1):
    c = lax.axis_index("c")
    s = lax.axis_index("s")
    w = c * NSUB + s
    pltpu.sync_copy(zrows.at[pl.ds(s * STRIPE, STRIPE)],
                    acc_sh.at[pl.ds(s * STRIPE, STRIPE)])
    plsc.subcore_barrier()

    for half in range(CH // BLK):
        pltpu.sync_copy(src3.at[w, pl.ds(half * BLK, BLK)], sidx)
        pltpu.sync_copy(dst3.at[w, pl.ds(half * BLK, BLK)], didx)
        pltpu.async_copy(m_hbm.at[sidx.at[0]], msg0, sem0)

        def body(t, carry):
            j = 2 * t
            pltpu.async_copy(m_hbm.at[sidx.at[j + 1]], msg1, sem1)
            pltpu.make_async_copy(m_hbm.at[sidx.at[j]], msg0, sem0).wait()
            pltpu.sync_copy(msg0, acc_sh.at[didx.at[j]], add=True)

            @pl.when(t < BLK // 2 - 1)
            def _prefetch():
                pltpu.async_copy(m_hbm.at[sidx.at[j + 2]], msg0, sem0)

            pltpu.make_async_copy(m_hbm.at[sidx.at[j + 1]], msg1, sem1).wait()
            pltpu.sync_copy(msg1, acc_sh.at[didx.at[j + 1]], add=True)
            return carry

        lax.fori_loop(0, BLK // 2, body, 0)
    plsc.subcore_barrier()
    pltpu.sync_copy(acc_sh.at[pl.ds(s * STRIPE, STRIPE)],
                    out.at[c, pl.ds(s * STRIPE, STRIPE)])


_sc_propagate = _make_propagate(D)


@functools.partial(
    pl.kernel,
    mesh=_mesh,
    out_type=jax.ShapeDtypeStruct((NCORE, NPAD * 8), jnp.float32),
    scratch_types=[
        pltpu.VMEM((CH, K), jnp.int32),
        pltpu.VMEM((CH, K), jnp.int32),
        pltpu.VMEM((K,), jnp.int32),
        pltpu.VMEM((K,), jnp.float32),
        pltpu.VMEM((K,), jnp.int32),
        pltpu.VMEM_SHARED((NPAD,), jnp.float32),
        pltpu.VMEM_SHARED((NPAD,), jnp.int32),
        pltpu.VMEM_SHARED((NPAD * 8,), jnp.float32),
    ],
)
def _sc_rank8(dinv_hbm, nt_hbm, src3, dst3, z8, out,
              sidx, didx, t_v, val_v, idx2, dinv_sh, nt_sh, acc_sh):
    c = lax.axis_index("c")
    s = lax.axis_index("s")
    w = c * NSUB + s
    pltpu.sync_copy(src3.at[w], sidx)
    pltpu.sync_copy(dst3.at[w], didx)
    pltpu.sync_copy(dinv_hbm.at[pl.ds(s * STRIPE, STRIPE)],
                    dinv_sh.at[pl.ds(s * STRIPE, STRIPE)])
    pltpu.sync_copy(nt_hbm.at[pl.ds(s * STRIPE, STRIPE)],
                    nt_sh.at[pl.ds(s * STRIPE, STRIPE)])
    pltpu.sync_copy(z8.at[pl.ds(s * STRIPE * 8, STRIPE * 8)],
                    acc_sh.at[pl.ds(s * STRIPE * 8, STRIPE * 8)])
    plsc.subcore_barrier()

    def body(j, carry):
        pltpu.sync_copy(dinv_sh.at[sidx.at[j]], val_v)
        pltpu.sync_copy(nt_sh.at[sidx.at[j]], t_v)
        for i in range(K // 16):
            sl = pl.ds(i * 16, 16)
            idx2[sl] = didx[j, sl] * 8 + t_v[sl]
        pltpu.sync_copy(val_v, acc_sh.at[idx2], add=True)
        return carry

    lax.fori_loop(0, CH, body, 0)
    plsc.subcore_barrier()
    pltpu.sync_copy(acc_sh.at[pl.ds(s * STRIPE * 8, STRIPE * 8)],
                    out.at[c, pl.ds(s * STRIPE * 8, STRIPE * 8)])


def _dinvs(d0_ref, d1_ref):
    deg = d0_ref[...] + d1_ref[...] + 1.0
    return lax.rsqrt(deg), 1.0 / deg


def _prep_body(d0_ref, d1_ref, nt_ref, emb_ref, w0_ref, dv_ref, s_ref):
    dinv, dinv2 = _dinvs(d0_ref, d1_ref)
    table0 = jnp.dot(emb_ref[...], w0_ref[...],
                     preferred_element_type=jnp.float32)
    nt = nt_ref[...]
    x0 = jnp.zeros((R, D), jnp.float32)
    for t in range(4):
        x0 += jnp.where(nt[:, None] == t, 1.0, 0.0) * table0[t, :][None, :]
    dv_ref[...] = dinv
    s_ref[...] = x0 * dinv2[:, None]


def _mid1_body(d0_ref, d1_ref, q_ref, s_ref, b_ref, emb_ref, w0_ref, w_ref,
               m_ref, so_ref):
    dinv, dinv2 = _dinvs(d0_ref, d1_ref)
    table0 = jnp.dot(emb_ref[...], w0_ref[...],
                     preferred_element_type=jnp.float32)
    q = q_ref[0] + q_ref[1]
    agg = jnp.zeros((R, D), jnp.float32)
    for t in range(4):
        agg += q[:, t][:, None] * table0[t, :][None, :]
    tot = dinv[:, None] * agg + s_ref[...] + b_ref[...][None, :]
    x = jnp.maximum(tot, 0.0)
    h = jnp.dot(x, w_ref[...], preferred_element_type=jnp.float32)
    m_ref[...] = h * dinv[:, None]
    so_ref[...] = h * dinv2[:, None]


def _mid_body(d0_ref, d1_ref, p_ref, s_ref, b_ref, w_ref, m_ref, so_ref):
    dinv, dinv2 = _dinvs(d0_ref, d1_ref)
    tot = dinv[:, None] * (p_ref[0] + p_ref[1]) + s_ref[...] + b_ref[...][None, :]
    x = jnp.maximum(tot, 0.0)
    h = jnp.dot(x, w_ref[...], preferred_element_type=jnp.float32)
    m_ref[...] = h * dinv[:, None]
    so_ref[...] = h * dinv2[:, None]


def _head_body(d0_ref, d1_ref, p_ref, s_ref, b_ref, w1_ref, b1_ref, w2_ref,
               b2_ref, out_ref):
    dinv, _ = _dinvs(d0_ref, d1_ref)
    tot = dinv[:, None] * (p_ref[0] + p_ref[1]) + s_ref[...] + b_ref[...][None, :]
    x = jnp.maximum(tot, 0.0)
    hh = jnp.dot(x, w1_ref[...], preferred_element_type=jnp.float32)
    hh = jnp.maximum(hh + b1_ref[...][None, :], 0.0)
    out_ref[...] = (jnp.dot(hh, w2_ref[...], preferred_element_type=jnp.float32)
                    + b2_ref[...][None, :])


def _tc_prep(d0, d1, node_type, node_emb, w0):
    return pl.pallas_call(
        _prep_body,
        grid=(NPAD // R,),
        in_specs=[
            pl.BlockSpec((R,), lambda i: (i,)),
            pl.BlockSpec((R,), lambda i: (i,)),
            pl.BlockSpec((R,), lambda i: (i,)),
            pl.BlockSpec((4, D), lambda i: (0, 0)),
            pl.BlockSpec((D, D), lambda i: (0, 0)),
        ],
        out_specs=[
            pl.BlockSpec((R,), lambda i: (i,)),
            pl.BlockSpec((R, D), lambda i: (i, 0)),
        ],
        out_shape=[
            jax.ShapeDtypeStruct((NPAD,), jnp.float32),
            jax.ShapeDtypeStruct((NPAD, D), jnp.float32),
        ],
    )(d0, d1, node_type, node_emb, w0)


def _tc_mid1(d0, d1, qp, sterm, b, node_emb, w0, w):
    return pl.pallas_call(
        _mid1_body,
        grid=(NPAD // R,),
        in_specs=[
            pl.BlockSpec((R,), lambda i: (i,)),
            pl.BlockSpec((R,), lambda i: (i,)),
            pl.BlockSpec((2, R, 8), lambda i: (0, i, 0)),
            pl.BlockSpec((R, D), lambda i: (i, 0)),
            pl.BlockSpec((D,), lambda i: (0,)),
            pl.BlockSpec((4, D), lambda i: (0, 0)),
            pl.BlockSpec((D, D), lambda i: (0, 0)),
            pl.BlockSpec((D, D), lambda i: (0, 0)),
        ],
        out_specs=[
            pl.BlockSpec((R, D), lambda i: (i, 0)),
            pl.BlockSpec((R, D), lambda i: (i, 0)),
        ],
        out_shape=[
            jax.ShapeDtypeStruct((NPAD, D), jnp.float32),
            jax.ShapeDtypeStruct((NPAD, D), jnp.float32),
        ],
    )(d0, d1, qp, sterm, b, node_emb, w0, w)


def _tc_mid(d0, d1, p, sterm, b, w):
    return pl.pallas_call(
        _mid_body,
        grid=(NPAD // R,),
        in_specs=[
            pl.BlockSpec((R,), lambda i: (i,)),
            pl.BlockSpec((R,), lambda i: (i,)),
            pl.BlockSpec((2, R, D), lambda i: (0, i, 0)),
            pl.BlockSpec((R, D), lambda i: (i, 0)),
            pl.BlockSpec((D,), lambda i: (0,)),
            pl.BlockSpec((D, D), lambda i: (0, 0)),
        ],
        out_specs=[
            pl.BlockSpec((R, D), lambda i: (i, 0)),
            pl.BlockSpec((R, D), lambda i: (i, 0)),
        ],
        out_shape=[
            jax.ShapeDtypeStruct((NPAD, D), jnp.float32),
            jax.ShapeDtypeStruct((NPAD, D), jnp.float32),
        ],
    )(d0, d1, p, sterm, b, w)


def _tc_head(d0, d1, p, sterm, b, w1, b1, w2, b2):
    return pl.pallas_call(
        _head_body,
        grid=(NPAD // R,),
        in_specs=[
            pl.BlockSpec((R,), lambda i: (i,)),
            pl.BlockSpec((R,), lambda i: (i,)),
            pl.BlockSpec((2, R, D), lambda i: (0, i, 0)),
            pl.BlockSpec((R, D), lambda i: (i, 0)),
            pl.BlockSpec((D,), lambda i: (0,)),
            pl.BlockSpec((D, D), lambda i: (0, 0)),
            pl.BlockSpec((D,), lambda i: (0,)),
            pl.BlockSpec((D, C), lambda i: (0, 0)),
            pl.BlockSpec((C,), lambda i: (0,)),
        ],
        out_specs=pl.BlockSpec((R, C), lambda i: (i, 0)),
        out_shape=jax.ShapeDtypeStruct((NPAD, C), jnp.float32),
    )(d0, d1, p, sterm, b, w1, b1, w2, b2)


def kernel(node_type, edge_type, edge_index, node_emb, edge_emb,
           gcn_W, gcn_b, head_W1, head_b1, head_W2, head_b2):
    del edge_type, edge_emb
    npad_e = EPAD - E
    pad_src = jnp.arange(npad_e, dtype=jnp.int32) % N
    pad_dst = N + jnp.arange(npad_e, dtype=jnp.int32) % (NPAD - N)
    src3 = jnp.concatenate([edge_index[0].astype(jnp.int32), pad_src]
                           ).reshape(NW, CH, K)
    dst3 = jnp.concatenate([edge_index[1].astype(jnp.int32), pad_dst]
                           ).reshape(NW, CH, K)
    z1d = jnp.zeros((NPAD,), jnp.float32)
    zrows = jnp.zeros((NPAD, D), jnp.float32)
    nt_pad = jnp.pad(node_type.astype(jnp.int32), (0, NPAD - N))

    degp = _sc_degree(dst3, z1d)
    d0, d1 = degp[0], degp[1]

    dinv_arr, sterm = _tc_prep(d0, d1, nt_pad, node_emb, gcn_W[0])
    z8 = jnp.zeros((NPAD * 8,), jnp.float32)
    qp = _sc_rank8(dinv_arr, nt_pad, src3, dst3, z8)
    m, sterm = _tc_mid1(d0, d1, qp.reshape(NCORE, NPAD, 8), sterm,
                        gcn_b[0], node_emb, gcn_W[0], gcn_W[1])
    for i in range(1, NUM_LAYERS):
        p = _sc_propagate(m, src3, dst3, zrows)
        if i < NUM_LAYERS - 1:
            m, sterm = _tc_mid(d0, d1, p, sterm, gcn_b[i], gcn_W[i + 1])
        else:
            out = _tc_head(d0, d1, p, sterm, gcn_b[i],
                           head_W1, head_b1, head_W2, head_b2)
    return out[:N]


# speedup vs baseline: 1.0084x; 1.0084x over previous
import functools

import jax
import jax.numpy as jnp
from jax import lax
from jax.experimental import pallas as pl
from jax.experimental.pallas import tpu as pltpu
from jax.experimental.pallas import tpu_sc as plsc

N = 10000
E = 320000
D = 128
C = 8
NUM_LAYERS = 3

NCORE = 2
NSUB = 16
NW = NCORE * NSUB
K = 128
CH = 80
EPAD = NW * CH * K
BLK = 40
NPAD = 10240
STRIPE = NPAD // NSUB

R = 2048

_mesh = plsc.VectorSubcoreMesh(core_axis_name="c", subcore_axis_name="s")


@functools.partial(
    pl.kernel,
    mesh=_mesh,
    out_type=jax.ShapeDtypeStruct((NCORE, NPAD), jnp.float32),
    scratch_types=[
        pltpu.VMEM((CH, K), jnp.int32),
        pltpu.VMEM((K,), jnp.float32),
        pltpu.VMEM_SHARED((NPAD,), jnp.float32),
    ],
)
def _sc_degree(dst3, z1d, out, didx, ones_v, deg_sh):
    c = lax.axis_index("c")
    s = lax.axis_index("s")
    w = c * NSUB + s
    pltpu.sync_copy(dst3.at[w], didx)
    one16 = jnp.ones((16,), jnp.float32)
    for i in range(K // 16):
        ones_v[pl.ds(i * 16, 16)] = one16
    pltpu.sync_copy(z1d.at[pl.ds(s * STRIPE, STRIPE)],
                    deg_sh.at[pl.ds(s * STRIPE, STRIPE)])
    plsc.subcore_barrier()

    def body(j, carry):
        pltpu.sync_copy(ones_v, deg_sh.at[didx.at[j]], add=True)
        return carry

    lax.fori_loop(0, CH, body, 0)
    plsc.subcore_barrier()
    pltpu.sync_copy(deg_sh.at[pl.ds(s * STRIPE, STRIPE)],
                    out.at[c, pl.ds(s * STRIPE, STRIPE)])


def _make_propagate(fd):
    @functools.partial(
        pl.kernel,
        mesh=_mesh,
        out_type=jax.ShapeDtypeStruct((NCORE, NPAD, fd), jnp.float32),
        scratch_types=[
            pltpu.VMEM((BLK, K), jnp.int32),
            pltpu.VMEM((BLK, K), jnp.int32),
            pltpu.VMEM((K, fd), jnp.float32),
            pltpu.VMEM((K, fd), jnp.float32),
            pltpu.VMEM_SHARED((NPAD, fd), jnp.float32),
            pltpu.SemaphoreType.DMA,
            pltpu.SemaphoreType.DMA,
        ],
    )
    def _prop(m_hbm, src3, dst3, zrows, out, sidx, didx, msg0, msg1,
              acc_sh, sem0, sem1):
        _propagate_body(m_hbm, src3, dst3, zrows, out, sidx, didx, msg0, msg1,
                        acc_sh, sem0, sem1)

    return _prop


def _propagate_body(m_hbm, src3, dst3, zrows, out, sidx, didx, msg0, msg1,
                    acc_sh, sem0, sem1):
    c = lax.axis_index("c")
    s = lax.axis_index("s")
    w = c * NSUB + s
    pltpu.sync_copy(zrows.at[pl.ds(s * STRIPE, STRIPE)],
                    acc_sh.at[pl.ds(s * STRIPE, STRIPE)])
    plsc.subcore_barrier()

    for half in range(CH // BLK):
        pltpu.sync_copy(src3.at[w, pl.ds(half * BLK, BLK)], sidx)
        pltpu.sync_copy(dst3.at[w, pl.ds(half * BLK, BLK)], didx)
        pltpu.async_copy(m_hbm.at[sidx.at[0]], msg0, sem0)

        def body(t, carry):
            j = 2 * t
            pltpu.async_copy(m_hbm.at[sidx.at[j + 1]], msg1, sem1)
            pltpu.make_async_copy(m_hbm.at[sidx.at[j]], msg0, sem0).wait()
            pltpu.sync_copy(msg0, acc_sh.at[didx.at[j]], add=True)

            @pl.when(t < BLK // 2 - 1)
            def _prefetch():
                pltpu.async_copy(m_hbm.at[sidx.at[j + 2]], msg0, sem0)

            pltpu.make_async_copy(m_hbm.at[sidx.at[j + 1]], msg1, sem1).wait()
            pltpu.sync_copy(msg1, acc_sh.at[didx.at[j + 1]], add=True)
            return carry

        lax.fori_loop(0, BLK // 2, body, 0)
    plsc.subcore_barrier()
    pltpu.sync_copy(acc_sh.at[pl.ds(s * STRIPE, STRIPE)],
                    out.at[c, pl.ds(s * STRIPE, STRIPE)])


_sc_propagate = _make_propagate(D)


@functools.partial(
    pl.kernel,
    mesh=_mesh,
    out_type=jax.ShapeDtypeStruct((NCORE, NPAD * 8), jnp.float32),
    scratch_types=[
        pltpu.VMEM((CH, K), jnp.int32),
        pltpu.VMEM((CH, K), jnp.int32),
        pltpu.VMEM((K,), jnp.int32),
        pltpu.VMEM((K,), jnp.float32),
        pltpu.VMEM((K,), jnp.int32),
        pltpu.VMEM_SHARED((NPAD,), jnp.float32),
        pltpu.VMEM_SHARED((NPAD,), jnp.int32),
        pltpu.VMEM_SHARED((NPAD * 8,), jnp.float32),
    ],
)
def _sc_rank8(dinv_hbm, nt_hbm, src3, dst3, z8, out,
              sidx, didx, t_v, val_v, idx2, dinv_sh, nt_sh, acc_sh):
    c = lax.axis_index("c")
    s = lax.axis_index("s")
    w = c * NSUB + s
    pltpu.sync_copy(src3.at[w], sidx)
    pltpu.sync_copy(dst3.at[w], didx)
    pltpu.sync_copy(dinv_hbm.at[pl.ds(s * STRIPE, STRIPE)],
                    dinv_sh.at[pl.ds(s * STRIPE, STRIPE)])
    pltpu.sync_copy(nt_hbm.at[pl.ds(s * STRIPE, STRIPE)],
                    nt_sh.at[pl.ds(s * STRIPE, STRIPE)])
    pltpu.sync_copy(z8.at[pl.ds(s * STRIPE * 8, STRIPE * 8)],
                    acc_sh.at[pl.ds(s * STRIPE * 8, STRIPE * 8)])
    plsc.subcore_barrier()

    def body(j, carry):
        pltpu.sync_copy(dinv_sh.at[sidx.at[j]], val_v)
        pltpu.sync_copy(nt_sh.at[sidx.at[j]], t_v)
        for i in range(K // 16):
            sl = pl.ds(i * 16, 16)
            idx2[sl] = didx[j, sl] * 8 + t_v[sl]
        pltpu.sync_copy(val_v, acc_sh.at[idx2], add=True)
        return carry

    lax.fori_loop(0, CH, body, 0)
    plsc.subcore_barrier()
    pltpu.sync_copy(acc_sh.at[pl.ds(s * STRIPE * 8, STRIPE * 8)],
                    out.at[c, pl.ds(s * STRIPE * 8, STRIPE * 8)])


def _dinvs(d0_ref, d1_ref):
    deg = d0_ref[...] + d1_ref[...] + 1.0
    return lax.rsqrt(deg), 1.0 / deg


def _dinv_body(d0_ref, d1_ref, dv_ref):
    dv_ref[...] = lax.rsqrt(d0_ref[...] + d1_ref[...] + 1.0)


def _tc_dinv(d0, d1):
    return pl.pallas_call(
        _dinv_body,
        grid=(NPAD // R,),
        in_specs=[
            pl.BlockSpec((R,), lambda i: (i,)),
            pl.BlockSpec((R,), lambda i: (i,)),
        ],
        out_specs=pl.BlockSpec((R,), lambda i: (i,)),
        out_shape=jax.ShapeDtypeStruct((NPAD,), jnp.float32),
    )(d0, d1)


def _prep_body(d0_ref, d1_ref, nt_ref, emb_ref, w0_ref, s_ref):
    _, dinv2 = _dinvs(d0_ref, d1_ref)
    table0 = jnp.dot(emb_ref[...], w0_ref[...],
                     preferred_element_type=jnp.float32)
    nt = nt_ref[...]
    x0 = jnp.zeros((R, D), jnp.float32)
    for t in range(4):
        x0 += jnp.where(nt[:, None] == t, 1.0, 0.0) * table0[t, :][None, :]
    s_ref[...] = x0 * dinv2[:, None]


def _mid1_body(d0_ref, d1_ref, q_ref, s_ref, b_ref, emb_ref, w0_ref, w_ref,
               m_ref, so_ref):
    dinv, dinv2 = _dinvs(d0_ref, d1_ref)
    table0 = jnp.dot(emb_ref[...], w0_ref[...],
                     preferred_element_type=jnp.float32)
    q = q_ref[0] + q_ref[1]
    agg = jnp.zeros((R, D), jnp.float32)
    for t in range(4):
        agg += q[:, t][:, None] * table0[t, :][None, :]
    tot = dinv[:, None] * agg + s_ref[...] + b_ref[...][None, :]
    x = jnp.maximum(tot, 0.0)
    h = jnp.dot(x, w_ref[...], preferred_element_type=jnp.float32)
    m_ref[...] = h * dinv[:, None]
    so_ref[...] = h * dinv2[:, None]


def _mid_body(d0_ref, d1_ref, p_ref, s_ref, b_ref, w_ref, m_ref, so_ref):
    dinv, dinv2 = _dinvs(d0_ref, d1_ref)
    tot = dinv[:, None] * (p_ref[0] + p_ref[1]) + s_ref[...] + b_ref[...][None, :]
    x = jnp.maximum(tot, 0.0)
    h = jnp.dot(x, w_ref[...], preferred_element_type=jnp.float32)
    m_ref[...] = h * dinv[:, None]
    so_ref[...] = h * dinv2[:, None]


def _head_body(d0_ref, d1_ref, p_ref, s_ref, b_ref, w1_ref, b1_ref, w2_ref,
               b2_ref, out_ref):
    dinv, _ = _dinvs(d0_ref, d1_ref)
    tot = dinv[:, None] * (p_ref[0] + p_ref[1]) + s_ref[...] + b_ref[...][None, :]
    x = jnp.maximum(tot, 0.0)
    hh = jnp.dot(x, w1_ref[...], preferred_element_type=jnp.float32)
    hh = jnp.maximum(hh + b1_ref[...][None, :], 0.0)
    out_ref[...] = (jnp.dot(hh, w2_ref[...], preferred_element_type=jnp.float32)
                    + b2_ref[...][None, :])


def _tc_prep(d0, d1, node_type, node_emb, w0):
    return pl.pallas_call(
        _prep_body,
        grid=(NPAD // R,),
        in_specs=[
            pl.BlockSpec((R,), lambda i: (i,)),
            pl.BlockSpec((R,), lambda i: (i,)),
            pl.BlockSpec((R,), lambda i: (i,)),
            pl.BlockSpec((4, D), lambda i: (0, 0)),
            pl.BlockSpec((D, D), lambda i: (0, 0)),
        ],
        out_specs=pl.BlockSpec((R, D), lambda i: (i, 0)),
        out_shape=jax.ShapeDtypeStruct((NPAD, D), jnp.float32),
    )(d0, d1, node_type, node_emb, w0)


def _tc_mid1(d0, d1, qp, sterm, b, node_emb, w0, w):
    return pl.pallas_call(
        _mid1_body,
        grid=(NPAD // R,),
        in_specs=[
            pl.BlockSpec((R,), lambda i: (i,)),
            pl.BlockSpec((R,), lambda i: (i,)),
            pl.BlockSpec((2, R, 8), lambda i: (0, i, 0)),
            pl.BlockSpec((R, D), lambda i: (i, 0)),
            pl.BlockSpec((D,), lambda i: (0,)),
            pl.BlockSpec((4, D), lambda i: (0, 0)),
            pl.BlockSpec((D, D), lambda i: (0, 0)),
            pl.BlockSpec((D, D), lambda i: (0, 0)),
        ],
        out_specs=[
            pl.BlockSpec((R, D), lambda i: (i, 0)),
            pl.BlockSpec((R, D), lambda i: (i, 0)),
        ],
        out_shape=[
            jax.ShapeDtypeStruct((NPAD, D), jnp.float32),
            jax.ShapeDtypeStruct((NPAD, D), jnp.float32),
        ],
    )(d0, d1, qp, sterm, b, node_emb, w0, w)


def _tc_mid(d0, d1, p, sterm, b, w):
    return pl.pallas_call(
        _mid_body,
        grid=(NPAD // R,),
        in_specs=[
            pl.BlockSpec((R,), lambda i: (i,)),
            pl.BlockSpec((R,), lambda i: (i,)),
            pl.BlockSpec((2, R, D), lambda i: (0, i, 0)),
            pl.BlockSpec((R, D), lambda i: (i, 0)),
            pl.BlockSpec((D,), lambda i: (0,)),
            pl.BlockSpec((D, D), lambda i: (0, 0)),
        ],
        out_specs=[
            pl.BlockSpec((R, D), lambda i: (i, 0)),
            pl.BlockSpec((R, D), lambda i: (i, 0)),
        ],
        out_shape=[
            jax.ShapeDtypeStruct((NPAD, D), jnp.float32),
            jax.ShapeDtypeStruct((NPAD, D), jnp.float32),
        ],
    )(d0, d1, p, sterm, b, w)


def _tc_head(d0, d1, p, sterm, b, w1, b1, w2, b2):
    return pl.pallas_call(
        _head_body,
        grid=(NPAD // R,),
        in_specs=[
            pl.BlockSpec((R,), lambda i: (i,)),
            pl.BlockSpec((R,), lambda i: (i,)),
            pl.BlockSpec((2, R, D), lambda i: (0, i, 0)),
            pl.BlockSpec((R, D), lambda i: (i, 0)),
            pl.BlockSpec((D,), lambda i: (0,)),
            pl.BlockSpec((D, D), lambda i: (0, 0)),
            pl.BlockSpec((D,), lambda i: (0,)),
            pl.BlockSpec((D, C), lambda i: (0, 0)),
            pl.BlockSpec((C,), lambda i: (0,)),
        ],
        out_specs=pl.BlockSpec((R, C), lambda i: (i, 0)),
        out_shape=jax.ShapeDtypeStruct((NPAD, C), jnp.float32),
    )(d0, d1, p, sterm, b, w1, b1, w2, b2)


def kernel(node_type, edge_type, edge_index, node_emb, edge_emb,
           gcn_W, gcn_b, head_W1, head_b1, head_W2, head_b2):
    del edge_type, edge_emb
    npad_e = EPAD - E
    pad_src = jnp.arange(npad_e, dtype=jnp.int32) % N
    pad_dst = N + jnp.arange(npad_e, dtype=jnp.int32) % (NPAD - N)
    src3 = jnp.concatenate([edge_index[0].astype(jnp.int32), pad_src]
                           ).reshape(NW, CH, K)
    dst3 = jnp.concatenate([edge_index[1].astype(jnp.int32), pad_dst]
                           ).reshape(NW, CH, K)
    z1d = jnp.zeros((NPAD,), jnp.float32)
    zrows = jnp.zeros((NPAD, D), jnp.float32)
    nt_pad = jnp.pad(node_type.astype(jnp.int32), (0, NPAD - N))

    degp = _sc_degree(dst3, z1d)
    d0, d1 = degp[0], degp[1]

    dinv_arr = _tc_dinv(d0, d1)
    sterm = _tc_prep(d0, d1, nt_pad, node_emb, gcn_W[0])
    z8 = jnp.zeros((NPAD * 8,), jnp.float32)
    qp = _sc_rank8(dinv_arr, nt_pad, src3, dst3, z8)
    m, sterm = _tc_mid1(d0, d1, qp.reshape(NCORE, NPAD, 8), sterm,
                        gcn_b[0], node_emb, gcn_W[0], gcn_W[1])
    for i in range(1, NUM_LAYERS):
        p = _sc_propagate(m, src3, dst3, zrows)
        if i < NUM_LAYERS - 1:
            m, sterm = _tc_mid(d0, d1, p, sterm, gcn_b[i], gcn_W[i + 1])
        else:
            out = _tc_head(d0, d1, p, sterm, gcn_b[i],
                           head_W1, head_b1, head_W2, head_b2)
    return out[:N]


# speedup vs baseline: 1.0365x; 1.0278x over previous
import functools

import jax
import jax.numpy as jnp
from jax import lax
from jax.experimental import pallas as pl
from jax.experimental.pallas import tpu as pltpu
from jax.experimental.pallas import tpu_sc as plsc

N = 10000
E = 320000
D = 128
C = 8
NUM_LAYERS = 3

NCORE = 2
NSUB = 16
NW = NCORE * NSUB
K = 128
CH = 80
EPAD = NW * CH * K
BLK = 40
NPAD = 10240
STRIPE = NPAD // NSUB

R = 2048

_mesh = plsc.VectorSubcoreMesh(core_axis_name="c", subcore_axis_name="s")


@functools.partial(
    pl.kernel,
    mesh=_mesh,
    out_type=jax.ShapeDtypeStruct((NCORE, NPAD), jnp.float32),
    scratch_types=[
        pltpu.VMEM((CH, K), jnp.int32),
        pltpu.VMEM((K,), jnp.float32),
        pltpu.VMEM_SHARED((NPAD,), jnp.float32),
    ],
)
def _sc_degree(dst3, z1d, out, didx, ones_v, deg_sh):
    c = lax.axis_index("c")
    s = lax.axis_index("s")
    w = c * NSUB + s
    pltpu.sync_copy(dst3.at[w], didx)
    one16 = jnp.ones((16,), jnp.float32)
    for i in range(K // 16):
        ones_v[pl.ds(i * 16, 16)] = one16
    pltpu.sync_copy(z1d.at[pl.ds(s * STRIPE, STRIPE)],
                    deg_sh.at[pl.ds(s * STRIPE, STRIPE)])
    plsc.subcore_barrier()

    def body(j, carry):
        pltpu.sync_copy(ones_v, deg_sh.at[didx.at[j]], add=True)
        return carry

    lax.fori_loop(0, CH, body, 0)
    plsc.subcore_barrier()
    pltpu.sync_copy(deg_sh.at[pl.ds(s * STRIPE, STRIPE)],
                    out.at[c, pl.ds(s * STRIPE, STRIPE)])


KB = 64
CHB = EPAD // NW // KB
BLKB = 40
DEPTH = 4


def _make_propagate(fd):
    @functools.partial(
        pl.kernel,
        mesh=_mesh,
        out_type=jax.ShapeDtypeStruct((NCORE, NPAD, fd), jnp.float32),
        scratch_types=[
            pltpu.VMEM((BLKB, KB), jnp.int32),
            pltpu.VMEM((BLKB, KB), jnp.int32),
            pltpu.VMEM((KB, fd), jnp.float32),
            pltpu.VMEM((KB, fd), jnp.float32),
            pltpu.VMEM((KB, fd), jnp.float32),
            pltpu.VMEM((KB, fd), jnp.float32),
            pltpu.VMEM_SHARED((NPAD, fd), jnp.float32),
            pltpu.SemaphoreType.DMA,
            pltpu.SemaphoreType.DMA,
            pltpu.SemaphoreType.DMA,
            pltpu.SemaphoreType.DMA,
        ],
    )
    def _prop(m_hbm, src3, dst3, zrows, out, sidx, didx, m0, m1, m2, m3,
              acc_sh, s0, s1, s2, s3):
        _propagate_body(m_hbm, src3, dst3, zrows, out, sidx, didx,
                        [m0, m1, m2, m3], acc_sh, [s0, s1, s2, s3])

    return _prop


def _propagate_body(m_hbm, src3, dst3, zrows, out, sidx, didx, msgs,
                    acc_sh, sems):
    c = lax.axis_index("c")
    s = lax.axis_index("s")
    w = c * NSUB + s
    pltpu.sync_copy(zrows.at[pl.ds(s * STRIPE, STRIPE)],
                    acc_sh.at[pl.ds(s * STRIPE, STRIPE)])
    plsc.subcore_barrier()

    for blk in range(CHB // BLKB):
        pltpu.sync_copy(src3.at[w, pl.ds(blk * BLKB, BLKB)], sidx)
        pltpu.sync_copy(dst3.at[w, pl.ds(blk * BLKB, BLKB)], didx)
        for b in range(DEPTH):
            pltpu.async_copy(m_hbm.at[sidx.at[b]], msgs[b], sems[b])

        def body(t, carry):
            for b in range(DEPTH):
                jj = DEPTH * t + b
                pltpu.make_async_copy(m_hbm.at[sidx.at[jj]],
                                      msgs[b], sems[b]).wait()
                pltpu.sync_copy(msgs[b], acc_sh.at[didx.at[jj]], add=True)

                @pl.when(t < BLKB // DEPTH - 1)
                def _prefetch(b=b, jj=jj):
                    pltpu.async_copy(m_hbm.at[sidx.at[jj + DEPTH]],
                                     msgs[b], sems[b])
            return carry

        lax.fori_loop(0, BLKB // DEPTH, body, 0)
    plsc.subcore_barrier()
    pltpu.sync_copy(acc_sh.at[pl.ds(s * STRIPE, STRIPE)],
                    out.at[c, pl.ds(s * STRIPE, STRIPE)])


_sc_propagate = _make_propagate(D)


@functools.partial(
    pl.kernel,
    mesh=_mesh,
    out_type=jax.ShapeDtypeStruct((NCORE, NPAD * 8), jnp.float32),
    scratch_types=[
        pltpu.VMEM((CH, K), jnp.int32),
        pltpu.VMEM((CH, K), jnp.int32),
        pltpu.VMEM((K,), jnp.int32),
        pltpu.VMEM((K,), jnp.float32),
        pltpu.VMEM((K,), jnp.int32),
        pltpu.VMEM_SHARED((NPAD,), jnp.float32),
        pltpu.VMEM_SHARED((NPAD,), jnp.int32),
        pltpu.VMEM_SHARED((NPAD * 8,), jnp.float32),
    ],
)
def _sc_rank8(dinv_hbm, nt_hbm, src3, dst3, z8, out,
              sidx, didx, t_v, val_v, idx2, dinv_sh, nt_sh, acc_sh):
    c = lax.axis_index("c")
    s = lax.axis_index("s")
    w = c * NSUB + s
    pltpu.sync_copy(src3.at[w], sidx)
    pltpu.sync_copy(dst3.at[w], didx)
    pltpu.sync_copy(dinv_hbm.at[pl.ds(s * STRIPE, STRIPE)],
                    dinv_sh.at[pl.ds(s * STRIPE, STRIPE)])
    pltpu.sync_copy(nt_hbm.at[pl.ds(s * STRIPE, STRIPE)],
                    nt_sh.at[pl.ds(s * STRIPE, STRIPE)])
    pltpu.sync_copy(z8.at[pl.ds(s * STRIPE * 8, STRIPE * 8)],
                    acc_sh.at[pl.ds(s * STRIPE * 8, STRIPE * 8)])
    plsc.subcore_barrier()

    def body(j, carry):
        pltpu.sync_copy(dinv_sh.at[sidx.at[j]], val_v)
        pltpu.sync_copy(nt_sh.at[sidx.at[j]], t_v)
        for i in range(K // 16):
            sl = pl.ds(i * 16, 16)
            idx2[sl] = didx[j, sl] * 8 + t_v[sl]
        pltpu.sync_copy(val_v, acc_sh.at[idx2], add=True)
        return carry

    lax.fori_loop(0, CH, body, 0)
    plsc.subcore_barrier()
    pltpu.sync_copy(acc_sh.at[pl.ds(s * STRIPE * 8, STRIPE * 8)],
                    out.at[c, pl.ds(s * STRIPE * 8, STRIPE * 8)])


def _dinvs(d0_ref, d1_ref):
    deg = d0_ref[...] + d1_ref[...] + 1.0
    return lax.rsqrt(deg), 1.0 / deg


def _dinv_body(d0_ref, d1_ref, dv_ref):
    dv_ref[...] = lax.rsqrt(d0_ref[...] + d1_ref[...] + 1.0)


def _tc_dinv(d0, d1):
    return pl.pallas_call(
        _dinv_body,
        grid=(NPAD // R,),
        in_specs=[
            pl.BlockSpec((R,), lambda i: (i,)),
            pl.BlockSpec((R,), lambda i: (i,)),
        ],
        out_specs=pl.BlockSpec((R,), lambda i: (i,)),
        out_shape=jax.ShapeDtypeStruct((NPAD,), jnp.float32),
    )(d0, d1)


def _prep_body(d0_ref, d1_ref, nt_ref, emb_ref, w0_ref, s_ref):
    _, dinv2 = _dinvs(d0_ref, d1_ref)
    table0 = jnp.dot(emb_ref[...], w0_ref[...],
                     preferred_element_type=jnp.float32)
    nt = nt_ref[...]
    x0 = jnp.zeros((R, D), jnp.float32)
    for t in range(4):
        x0 += jnp.where(nt[:, None] == t, 1.0, 0.0) * table0[t, :][None, :]
    s_ref[...] = x0 * dinv2[:, None]


def _mid1_body(d0_ref, d1_ref, q_ref, s_ref, b_ref, emb_ref, w0_ref, w_ref,
               m_ref, so_ref):
    dinv, dinv2 = _dinvs(d0_ref, d1_ref)
    table0 = jnp.dot(emb_ref[...], w0_ref[...],
                     preferred_element_type=jnp.float32)
    q = q_ref[0] + q_ref[1]
    agg = jnp.zeros((R, D), jnp.float32)
    for t in range(4):
        agg += q[:, t][:, None] * table0[t, :][None, :]
    tot = dinv[:, None] * agg + s_ref[...] + b_ref[...][None, :]
    x = jnp.maximum(tot, 0.0)
    h = jnp.dot(x, w_ref[...], preferred_element_type=jnp.float32)
    m_ref[...] = h * dinv[:, None]
    so_ref[...] = h * dinv2[:, None]


def _mid_body(d0_ref, d1_ref, p_ref, s_ref, b_ref, w_ref, m_ref, so_ref):
    dinv, dinv2 = _dinvs(d0_ref, d1_ref)
    tot = dinv[:, None] * (p_ref[0] + p_ref[1]) + s_ref[...] + b_ref[...][None, :]
    x = jnp.maximum(tot, 0.0)
    h = jnp.dot(x, w_ref[...], preferred_element_type=jnp.float32)
    m_ref[...] = h * dinv[:, None]
    so_ref[...] = h * dinv2[:, None]


def _head_body(d0_ref, d1_ref, p_ref, s_ref, b_ref, w1_ref, b1_ref, w2_ref,
               b2_ref, out_ref):
    dinv, _ = _dinvs(d0_ref, d1_ref)
    tot = dinv[:, None] * (p_ref[0] + p_ref[1]) + s_ref[...] + b_ref[...][None, :]
    x = jnp.maximum(tot, 0.0)
    hh = jnp.dot(x, w1_ref[...], preferred_element_type=jnp.float32)
    hh = jnp.maximum(hh + b1_ref[...][None, :], 0.0)
    out_ref[...] = (jnp.dot(hh, w2_ref[...], preferred_element_type=jnp.float32)
                    + b2_ref[...][None, :])


def _tc_prep(d0, d1, node_type, node_emb, w0):
    return pl.pallas_call(
        _prep_body,
        grid=(NPAD // R,),
        in_specs=[
            pl.BlockSpec((R,), lambda i: (i,)),
            pl.BlockSpec((R,), lambda i: (i,)),
            pl.BlockSpec((R,), lambda i: (i,)),
            pl.BlockSpec((4, D), lambda i: (0, 0)),
            pl.BlockSpec((D, D), lambda i: (0, 0)),
        ],
        out_specs=pl.BlockSpec((R, D), lambda i: (i, 0)),
        out_shape=jax.ShapeDtypeStruct((NPAD, D), jnp.float32),
    )(d0, d1, node_type, node_emb, w0)


def _tc_mid1(d0, d1, qp, sterm, b, node_emb, w0, w):
    return pl.pallas_call(
        _mid1_body,
        grid=(NPAD // R,),
        in_specs=[
            pl.BlockSpec((R,), lambda i: (i,)),
            pl.BlockSpec((R,), lambda i: (i,)),
            pl.BlockSpec((2, R, 8), lambda i: (0, i, 0)),
            pl.BlockSpec((R, D), lambda i: (i, 0)),
            pl.BlockSpec((D,), lambda i: (0,)),
            pl.BlockSpec((4, D), lambda i: (0, 0)),
            pl.BlockSpec((D, D), lambda i: (0, 0)),
            pl.BlockSpec((D, D), lambda i: (0, 0)),
        ],
        out_specs=[
            pl.BlockSpec((R, D), lambda i: (i, 0)),
            pl.BlockSpec((R, D), lambda i: (i, 0)),
        ],
        out_shape=[
            jax.ShapeDtypeStruct((NPAD, D), jnp.float32),
            jax.ShapeDtypeStruct((NPAD, D), jnp.float32),
        ],
    )(d0, d1, qp, sterm, b, node_emb, w0, w)


def _tc_mid(d0, d1, p, sterm, b, w):
    return pl.pallas_call(
        _mid_body,
        grid=(NPAD // R,),
        in_specs=[
            pl.BlockSpec((R,), lambda i: (i,)),
            pl.BlockSpec((R,), lambda i: (i,)),
            pl.BlockSpec((2, R, D), lambda i: (0, i, 0)),
            pl.BlockSpec((R, D), lambda i: (i, 0)),
            pl.BlockSpec((D,), lambda i: (0,)),
            pl.BlockSpec((D, D), lambda i: (0, 0)),
        ],
        out_specs=[
            pl.BlockSpec((R, D), lambda i: (i, 0)),
            pl.BlockSpec((R, D), lambda i: (i, 0)),
        ],
        out_shape=[
            jax.ShapeDtypeStruct((NPAD, D), jnp.float32),
            jax.ShapeDtypeStruct((NPAD, D), jnp.float32),
        ],
    )(d0, d1, p, sterm, b, w)


def _tc_head(d0, d1, p, sterm, b, w1, b1, w2, b2):
    return pl.pallas_call(
        _head_body,
        grid=(NPAD // R,),
        in_specs=[
            pl.BlockSpec((R,), lambda i: (i,)),
            pl.BlockSpec((R,), lambda i: (i,)),
            pl.BlockSpec((2, R, D), lambda i: (0, i, 0)),
            pl.BlockSpec((R, D), lambda i: (i, 0)),
            pl.BlockSpec((D,), lambda i: (0,)),
            pl.BlockSpec((D, D), lambda i: (0, 0)),
            pl.BlockSpec((D,), lambda i: (0,)),
            pl.BlockSpec((D, C), lambda i: (0, 0)),
            pl.BlockSpec((C,), lambda i: (0,)),
        ],
        out_specs=pl.BlockSpec((R, C), lambda i: (i, 0)),
        out_shape=jax.ShapeDtypeStruct((NPAD, C), jnp.float32),
    )(d0, d1, p, sterm, b, w1, b1, w2, b2)


def kernel(node_type, edge_type, edge_index, node_emb, edge_emb,
           gcn_W, gcn_b, head_W1, head_b1, head_W2, head_b2):
    del edge_type, edge_emb
    npad_e = EPAD - E
    pad_src = jnp.arange(npad_e, dtype=jnp.int32) % N
    pad_dst = N + jnp.arange(npad_e, dtype=jnp.int32) % (NPAD - N)
    src3 = jnp.concatenate([edge_index[0].astype(jnp.int32), pad_src]
                           ).reshape(NW, CH, K)
    dst3 = jnp.concatenate([edge_index[1].astype(jnp.int32), pad_dst]
                           ).reshape(NW, CH, K)
    z1d = jnp.zeros((NPAD,), jnp.float32)
    zrows = jnp.zeros((NPAD, D), jnp.float32)
    nt_pad = jnp.pad(node_type.astype(jnp.int32), (0, NPAD - N))

    degp = _sc_degree(dst3, z1d)
    d0, d1 = degp[0], degp[1]

    dinv_arr = _tc_dinv(d0, d1)
    sterm = _tc_prep(d0, d1, nt_pad, node_emb, gcn_W[0])
    z8 = jnp.zeros((NPAD * 8,), jnp.float32)
    qp = _sc_rank8(dinv_arr, nt_pad, src3, dst3, z8)
    m, sterm = _tc_mid1(d0, d1, qp.reshape(NCORE, NPAD, 8), sterm,
                        gcn_b[0], node_emb, gcn_W[0], gcn_W[1])
    srcp = src3.reshape(NW, CHB, KB)
    dstp = dst3.reshape(NW, CHB, KB)
    for i in range(1, NUM_LAYERS):
        p = _sc_propagate(m, srcp, dstp, zrows)
        if i < NUM_LAYERS - 1:
            m, sterm = _tc_mid(d0, d1, p, sterm, gcn_b[i], gcn_W[i + 1])
        else:
            out = _tc_head(d0, d1, p, sterm, gcn_b[i],
                           head_W1, head_b1, head_W2, head_b2)
    return out[:N]


# speedup vs baseline: 1.0972x; 1.0586x over previous
import functools

import jax
import jax.numpy as jnp
from jax import lax
from jax.experimental import pallas as pl
from jax.experimental.pallas import tpu as pltpu
from jax.experimental.pallas import tpu_sc as plsc

N = 10000
E = 320000
D = 128
C = 8
NUM_LAYERS = 3

NCORE = 2
NSUB = 16
NW = NCORE * NSUB
K = 128
CH = 80
EPAD = NW * CH * K
BLK = 40
NPAD = 10240
STRIPE = NPAD // NSUB

R = 2048

_mesh = plsc.VectorSubcoreMesh(core_axis_name="c", subcore_axis_name="s")


@functools.partial(
    pl.kernel,
    mesh=_mesh,
    out_type=jax.ShapeDtypeStruct((NCORE, NPAD), jnp.float32),
    scratch_types=[
        pltpu.VMEM((CH, K), jnp.int32),
        pltpu.VMEM((K,), jnp.float32),
        pltpu.VMEM_SHARED((NPAD,), jnp.float32),
    ],
)
def _sc_degree(dst3, z1d, out, didx, ones_v, deg_sh):
    c = lax.axis_index("c")
    s = lax.axis_index("s")
    w = c * NSUB + s
    pltpu.sync_copy(dst3.at[w], didx)
    one16 = jnp.ones((16,), jnp.float32)
    for i in range(K // 16):
        ones_v[pl.ds(i * 16, 16)] = one16
    pltpu.sync_copy(z1d.at[pl.ds(s * STRIPE, STRIPE)],
                    deg_sh.at[pl.ds(s * STRIPE, STRIPE)])
    plsc.subcore_barrier()

    def body(j, carry):
        pltpu.sync_copy(ones_v, deg_sh.at[didx.at[j]], add=True)
        return carry

    lax.fori_loop(0, CH, body, 0)
    plsc.subcore_barrier()
    pltpu.sync_copy(deg_sh.at[pl.ds(s * STRIPE, STRIPE)],
                    out.at[c, pl.ds(s * STRIPE, STRIPE)])


KB = 64
CHB = EPAD // NW // KB
BLKB = 40
DEPTH = 4


def _make_propagate(fd):
    @functools.partial(
        pl.kernel,
        mesh=_mesh,
        out_type=jax.ShapeDtypeStruct((NCORE, NPAD, fd), jnp.float32),
        scratch_types=[
            pltpu.VMEM((BLKB, KB), jnp.int32),
            pltpu.VMEM((BLKB, KB), jnp.int32),
            pltpu.VMEM((KB, fd), jnp.float32),
            pltpu.VMEM((KB, fd), jnp.float32),
            pltpu.VMEM((KB, fd), jnp.float32),
            pltpu.VMEM((KB, fd), jnp.float32),
            pltpu.VMEM_SHARED((NPAD, fd), jnp.float32),
            pltpu.SemaphoreType.DMA,
            pltpu.SemaphoreType.DMA,
            pltpu.SemaphoreType.DMA,
            pltpu.SemaphoreType.DMA,
        ],
    )
    def _prop(m_hbm, src3, dst3, zrows, out, sidx, didx, m0, m1, m2, m3,
              acc_sh, s0, s1, s2, s3):
        _propagate_body(m_hbm, src3, dst3, zrows, out, sidx, didx,
                        [m0, m1, m2, m3], acc_sh, [s0, s1, s2, s3])

    return _prop


def _propagate_body(m_hbm, src3, dst3, zrows, out, sidx, didx, msgs,
                    acc_sh, sems):
    c = lax.axis_index("c")
    s = lax.axis_index("s")
    w = c * NSUB + s
    pltpu.sync_copy(zrows.at[pl.ds(s * STRIPE, STRIPE)],
                    acc_sh.at[pl.ds(s * STRIPE, STRIPE)])
    plsc.subcore_barrier()

    for blk in range(CHB // BLKB):
        pltpu.sync_copy(src3.at[w, pl.ds(blk * BLKB, BLKB)], sidx)
        pltpu.sync_copy(dst3.at[w, pl.ds(blk * BLKB, BLKB)], didx)
        for b in range(DEPTH):
            pltpu.async_copy(m_hbm.at[sidx.at[b]], msgs[b], sems[b])

        def body(t, carry):
            for b in range(DEPTH):
                jj = DEPTH * t + b
                pltpu.make_async_copy(m_hbm.at[sidx.at[jj]],
                                      msgs[b], sems[b]).wait()
                pltpu.sync_copy(msgs[b], acc_sh.at[didx.at[jj]], add=True)

                @pl.when(t < BLKB // DEPTH - 1)
                def _prefetch(b=b, jj=jj):
                    pltpu.async_copy(m_hbm.at[sidx.at[jj + DEPTH]],
                                     msgs[b], sems[b])
            return carry

        lax.fori_loop(0, BLKB // DEPTH, body, 0)
    plsc.subcore_barrier()
    pltpu.sync_copy(acc_sh.at[pl.ds(s * STRIPE, STRIPE)],
                    out.at[c, pl.ds(s * STRIPE, STRIPE)])


_sc_propagate = _make_propagate(D)


@functools.partial(
    pl.kernel,
    mesh=_mesh,
    out_type=jax.ShapeDtypeStruct((NCORE, NPAD * 8), jnp.float32),
    scratch_types=[
        pltpu.VMEM((CH, K), jnp.int32),
        pltpu.VMEM((CH, K), jnp.int32),
        pltpu.VMEM((K,), jnp.int32),
        pltpu.VMEM((K,), jnp.int32),
        pltpu.VMEM((K,), jnp.float32),
        pltpu.VMEM((K,), jnp.float32),
        pltpu.VMEM((K,), jnp.int32),
        pltpu.VMEM_SHARED((NPAD,), jnp.float32),
        pltpu.VMEM_SHARED((NPAD,), jnp.int32),
        pltpu.VMEM_SHARED((NPAD * 8,), jnp.float32),
        pltpu.SemaphoreType.DMA,
        pltpu.SemaphoreType.DMA,
        pltpu.SemaphoreType.DMA,
        pltpu.SemaphoreType.DMA,
    ],
)
def _sc_rank8(dinv_hbm, nt_hbm, src3, dst3, z8, out,
              sidx, didx, t0, t1, val0, val1, idx2, dinv_sh, nt_sh, acc_sh,
              sv0, st0, sv1, st1):
    c = lax.axis_index("c")
    s = lax.axis_index("s")
    w = c * NSUB + s
    pltpu.sync_copy(src3.at[w], sidx)
    pltpu.sync_copy(dst3.at[w], didx)
    pltpu.sync_copy(dinv_hbm.at[pl.ds(s * STRIPE, STRIPE)],
                    dinv_sh.at[pl.ds(s * STRIPE, STRIPE)])
    pltpu.sync_copy(nt_hbm.at[pl.ds(s * STRIPE, STRIPE)],
                    nt_sh.at[pl.ds(s * STRIPE, STRIPE)])
    pltpu.sync_copy(z8.at[pl.ds(s * STRIPE * 8, STRIPE * 8)],
                    acc_sh.at[pl.ds(s * STRIPE * 8, STRIPE * 8)])
    plsc.subcore_barrier()

    pltpu.async_copy(dinv_sh.at[sidx.at[0]], val0, sv0)
    pltpu.async_copy(nt_sh.at[sidx.at[0]], t0, st0)

    def _do_chunk(j, val_v, t_v, sv, st):
        pltpu.make_async_copy(dinv_sh.at[sidx.at[j]], val_v, sv).wait()
        pltpu.make_async_copy(nt_sh.at[sidx.at[j]], t_v, st).wait()
        for i in range(K // 16):
            sl = pl.ds(i * 16, 16)
            idx2[sl] = didx[j, sl] * 8 + t_v[sl]
        pltpu.sync_copy(val_v, acc_sh.at[idx2], add=True)

    def body(u, carry):
        j = 2 * u
        pltpu.async_copy(dinv_sh.at[sidx.at[j + 1]], val1, sv1)
        pltpu.async_copy(nt_sh.at[sidx.at[j + 1]], t1, st1)
        _do_chunk(j, val0, t0, sv0, st0)

        @pl.when(u < CH // 2 - 1)
        def _prefetch():
            pltpu.async_copy(dinv_sh.at[sidx.at[j + 2]], val0, sv0)
            pltpu.async_copy(nt_sh.at[sidx.at[j + 2]], t0, st0)

        _do_chunk(j + 1, val1, t1, sv1, st1)
        return carry

    lax.fori_loop(0, CH // 2, body, 0)
    plsc.subcore_barrier()
    pltpu.sync_copy(acc_sh.at[pl.ds(s * STRIPE * 8, STRIPE * 8)],
                    out.at[c, pl.ds(s * STRIPE * 8, STRIPE * 8)])


def _dinvs(d0_ref, d1_ref):
    deg = d0_ref[...] + d1_ref[...] + 1.0
    return lax.rsqrt(deg), 1.0 / deg


def _dinv_body(d0_ref, d1_ref, dv_ref):
    dv_ref[...] = lax.rsqrt(d0_ref[...] + d1_ref[...] + 1.0)


def _tc_dinv(d0, d1):
    return pl.pallas_call(
        _dinv_body,
        grid=(NPAD // R,),
        in_specs=[
            pl.BlockSpec((R,), lambda i: (i,)),
            pl.BlockSpec((R,), lambda i: (i,)),
        ],
        out_specs=pl.BlockSpec((R,), lambda i: (i,)),
        out_shape=jax.ShapeDtypeStruct((NPAD,), jnp.float32),
    )(d0, d1)


def _prep_body(d0_ref, d1_ref, nt_ref, emb_ref, w0_ref, s_ref):
    _, dinv2 = _dinvs(d0_ref, d1_ref)
    table0 = jnp.dot(emb_ref[...], w0_ref[...],
                     preferred_element_type=jnp.float32)
    nt = nt_ref[...]
    x0 = jnp.zeros((R, D), jnp.float32)
    for t in range(4):
        x0 += jnp.where(nt[:, None] == t, 1.0, 0.0) * table0[t, :][None, :]
    s_ref[...] = x0 * dinv2[:, None]


def _mid1_body(d0_ref, d1_ref, q_ref, s_ref, b_ref, emb_ref, w0_ref, w_ref,
               m_ref, so_ref):
    dinv, dinv2 = _dinvs(d0_ref, d1_ref)
    table0 = jnp.dot(emb_ref[...], w0_ref[...],
                     preferred_element_type=jnp.float32)
    q = q_ref[0] + q_ref[1]
    agg = jnp.zeros((R, D), jnp.float32)
    for t in range(4):
        agg += q[:, t][:, None] * table0[t, :][None, :]
    tot = dinv[:, None] * agg + s_ref[...] + b_ref[...][None, :]
    x = jnp.maximum(tot, 0.0)
    h = jnp.dot(x, w_ref[...], preferred_element_type=jnp.float32)
    m_ref[...] = h * dinv[:, None]
    so_ref[...] = h * dinv2[:, None]


def _mid_body(d0_ref, d1_ref, p_ref, s_ref, b_ref, w_ref, m_ref, so_ref):
    dinv, dinv2 = _dinvs(d0_ref, d1_ref)
    tot = dinv[:, None] * (p_ref[0] + p_ref[1]) + s_ref[...] + b_ref[...][None, :]
    x = jnp.maximum(tot, 0.0)
    h = jnp.dot(x, w_ref[...], preferred_element_type=jnp.float32)
    m_ref[...] = h * dinv[:, None]
    so_ref[...] = h * dinv2[:, None]


def _head_body(d0_ref, d1_ref, p_ref, s_ref, b_ref, w1_ref, b1_ref, w2_ref,
               b2_ref, out_ref):
    dinv, _ = _dinvs(d0_ref, d1_ref)
    tot = dinv[:, None] * (p_ref[0] + p_ref[1]) + s_ref[...] + b_ref[...][None, :]
    x = jnp.maximum(tot, 0.0)
    hh = jnp.dot(x, w1_ref[...], preferred_element_type=jnp.float32)
    hh = jnp.maximum(hh + b1_ref[...][None, :], 0.0)
    out_ref[...] = (jnp.dot(hh, w2_ref[...], preferred_element_type=jnp.float32)
                    + b2_ref[...][None, :])


def _tc_prep(d0, d1, node_type, node_emb, w0):
    return pl.pallas_call(
        _prep_body,
        grid=(NPAD // R,),
        in_specs=[
            pl.BlockSpec((R,), lambda i: (i,)),
            pl.BlockSpec((R,), lambda i: (i,)),
            pl.BlockSpec((R,), lambda i: (i,)),
            pl.BlockSpec((4, D), lambda i: (0, 0)),
            pl.BlockSpec((D, D), lambda i: (0, 0)),
        ],
        out_specs=pl.BlockSpec((R, D), lambda i: (i, 0)),
        out_shape=jax.ShapeDtypeStruct((NPAD, D), jnp.float32),
    )(d0, d1, node_type, node_emb, w0)


def _tc_mid1(d0, d1, qp, sterm, b, node_emb, w0, w):
    return pl.pallas_call(
        _mid1_body,
        grid=(NPAD // R,),
        in_specs=[
            pl.BlockSpec((R,), lambda i: (i,)),
            pl.BlockSpec((R,), lambda i: (i,)),
            pl.BlockSpec((2, R, 8), lambda i: (0, i, 0)),
            pl.BlockSpec((R, D), lambda i: (i, 0)),
            pl.BlockSpec((D,), lambda i: (0,)),
            pl.BlockSpec((4, D), lambda i: (0, 0)),
            pl.BlockSpec((D, D), lambda i: (0, 0)),
            pl.BlockSpec((D, D), lambda i: (0, 0)),
        ],
        out_specs=[
            pl.BlockSpec((R, D), lambda i: (i, 0)),
            pl.BlockSpec((R, D), lambda i: (i, 0)),
        ],
        out_shape=[
            jax.ShapeDtypeStruct((NPAD, D), jnp.float32),
            jax.ShapeDtypeStruct((NPAD, D), jnp.float32),
        ],
    )(d0, d1, qp, sterm, b, node_emb, w0, w)


def _tc_mid(d0, d1, p, sterm, b, w):
    return pl.pallas_call(
        _mid_body,
        grid=(NPAD // R,),
        in_specs=[
            pl.BlockSpec((R,), lambda i: (i,)),
            pl.BlockSpec((R,), lambda i: (i,)),
            pl.BlockSpec((2, R, D), lambda i: (0, i, 0)),
            pl.BlockSpec((R, D), lambda i: (i, 0)),
            pl.BlockSpec((D,), lambda i: (0,)),
            pl.BlockSpec((D, D), lambda i: (0, 0)),
        ],
        out_specs=[
            pl.BlockSpec((R, D), lambda i: (i, 0)),
            pl.BlockSpec((R, D), lambda i: (i, 0)),
        ],
        out_shape=[
            jax.ShapeDtypeStruct((NPAD, D), jnp.float32),
            jax.ShapeDtypeStruct((NPAD, D), jnp.float32),
        ],
    )(d0, d1, p, sterm, b, w)


def _tc_head(d0, d1, p, sterm, b, w1, b1, w2, b2):
    return pl.pallas_call(
        _head_body,
        grid=(NPAD // R,),
        in_specs=[
            pl.BlockSpec((R,), lambda i: (i,)),
            pl.BlockSpec((R,), lambda i: (i,)),
            pl.BlockSpec((2, R, D), lambda i: (0, i, 0)),
            pl.BlockSpec((R, D), lambda i: (i, 0)),
            pl.BlockSpec((D,), lambda i: (0,)),
            pl.BlockSpec((D, D), lambda i: (0, 0)),
            pl.BlockSpec((D,), lambda i: (0,)),
            pl.BlockSpec((D, C), lambda i: (0, 0)),
            pl.BlockSpec((C,), lambda i: (0,)),
        ],
        out_specs=pl.BlockSpec((R, C), lambda i: (i, 0)),
        out_shape=jax.ShapeDtypeStruct((NPAD, C), jnp.float32),
    )(d0, d1, p, sterm, b, w1, b1, w2, b2)


def kernel(node_type, edge_type, edge_index, node_emb, edge_emb,
           gcn_W, gcn_b, head_W1, head_b1, head_W2, head_b2):
    del edge_type, edge_emb
    npad_e = EPAD - E
    pad_src = jnp.arange(npad_e, dtype=jnp.int32) % N
    pad_dst = N + jnp.arange(npad_e, dtype=jnp.int32) % (NPAD - N)
    src3 = jnp.concatenate([edge_index[0].astype(jnp.int32), pad_src]
                           ).reshape(NW, CH, K)
    dst3 = jnp.concatenate([edge_index[1].astype(jnp.int32), pad_dst]
                           ).reshape(NW, CH, K)
    z1d = jnp.zeros((NPAD,), jnp.float32)
    zrows = jnp.zeros((NPAD, D), jnp.float32)
    nt_pad = jnp.pad(node_type.astype(jnp.int32), (0, NPAD - N))

    degp = _sc_degree(dst3, z1d)
    d0, d1 = degp[0], degp[1]

    dinv_arr = _tc_dinv(d0, d1)
    sterm = _tc_prep(d0, d1, nt_pad, node_emb, gcn_W[0])
    z8 = jnp.zeros((NPAD * 8,), jnp.float32)
    qp = _sc_rank8(dinv_arr, nt_pad, src3, dst3, z8)
    m, sterm = _tc_mid1(d0, d1, qp.reshape(NCORE, NPAD, 8), sterm,
                        gcn_b[0], node_emb, gcn_W[0], gcn_W[1])
    srcp = src3.reshape(NW, CHB, KB)
    dstp = dst3.reshape(NW, CHB, KB)
    for i in range(1, NUM_LAYERS):
        p = _sc_propagate(m, srcp, dstp, zrows)
        if i < NUM_LAYERS - 1:
            m, sterm = _tc_mid(d0, d1, p, sterm, gcn_b[i], gcn_W[i + 1])
        else:
            out = _tc_head(d0, d1, p, sterm, gcn_b[i],
                           head_W1, head_b1, head_W2, head_b2)
    return out[:N]
